# baseline (speedup 1.0000x reference)
import jax
import jax.numpy as jnp
from jax import lax
from jax.experimental import pallas as pl
from jax.experimental.pallas import tpu as pltpu
from jax.experimental.pallas import tpu_sc as plsc

_B = 16384
_D = 128
_K = _D // 16
_NC = 2
_NS = 16
_NW = _NC * _NS
_RPW = _B // _NW
_C = 64
_NCH = _RPW // _C
_CAT = 5 * _C
_OFFS = (0, 25, 125, 525, 1225)


def _rsqrt_nr(x):
    xi = lax.bitcast_convert_type(x, jnp.int32)
    yi = jnp.int32(0x5F3759DF) - lax.shift_right_logical(xi, 1)
    y = lax.bitcast_convert_type(yi, jnp.float32)
    hx = x * jnp.float32(0.5)
    for _ in range(2):
        y = y * (jnp.float32(1.5) - hx * y * y)
    return y


def _splat(s):
    return lax.broadcast_in_dim(s, (16,), ())


def _body(i2, i3, i4, i5, i6, tcat, wb,
          out_hbm,
          ix0, ix1, ix2, ix3, ix4, ixc,
          gc0, gc1, out_v, wb_v, semA, semB):
    wid = lax.axis_index("s") * _NC + lax.axis_index("c")
    base = wid * _RPW

    idx_refs = (ix0, ix1, ix2, ix3, ix4)
    for idx_hbm, idx_v in zip((i2, i3, i4, i5, i6), idx_refs):
        pltpu.sync_copy(idx_hbm.at[pl.ds(base, _RPW)], idx_v)
    pltpu.sync_copy(wb, wb_v)

    for l in range(5):
        off = jnp.full((16,), _OFFS[l], jnp.int32)
        src = idx_refs[l]

        def mk_cat(i, off=off, src=src):
            ch = i // 4
            q = i % 4
            dst = pl.multiple_of(ch * _CAT + l * _C + q * 16, 16)
            sp = pl.multiple_of(i * 16, 16)
            ixc[pl.ds(dst, 16)] = src[pl.ds(sp, 16)] + off

        plsc.parallel_loop(0, _RPW // 16)(mk_cat)

    gbufs = (gc0, gc1)
    sems = (semA, semB)

    w_regs = [wb_v[pl.ds(k * 16, 16)] for k in range(_K)]
    b_splat = _splat(wb_v[pl.ds(_D, 16)][0])
    lane0 = lax.broadcasted_iota(jnp.int32, (16,), 0) == 0

    def issue(ch):
        p = ch % 2
        return pltpu.async_copy(
            tcat.at[ixc.at[pl.ds(ch * _CAT, _CAT)]], gbufs[p], sems[p])

    pending = issue(0)
    for ch in range(_NCH):
        pending.wait()
        if ch + 1 < _NCH:
            pending = issue(ch + 1)
        g = gbufs[ch % 2]
        cb = ch * _C

        def row_body(r, _, g=g, cb=cb):
            u = [g[r, pl.ds(k * 16, 16)] for k in range(_K)]
            acc = u[0] * u[0]
            for k in range(1, _K):
                acc = acc + u[k] * u[k]
            y = _rsqrt_nr(_splat(jnp.sum(acc)))
            for l in range(1, 5):
                u = [y * u[k] + g[l * _C + r, pl.ds(k * 16, 16)]
                     for k in range(_K)]
                acc = u[0] * u[0]
                for k in range(1, _K):
                    acc = acc + u[k] * u[k]
                y = _rsqrt_nr(_splat(jnp.sum(acc)))
            dotv = u[0] * w_regs[0]
            for k in range(1, _K):
                dotv = dotv + u[k] * w_regs[k]
            row_out = y * _splat(jnp.sum(dotv)) + b_splat
            plsc.store_scatter(out_v, [jnp.full((16,), cb + r, jnp.int32)],
                               row_out, mask=lane0)
            return 0

        plsc.parallel_loop(0, _C, carry=jnp.int32(0))(row_body)

    pltpu.sync_copy(out_v, out_hbm.at[pl.ds(base, _RPW)])


def kernel(naics_2_digit, naics_3_digit, naics_4_digit, naics_5_digit, naics_6_digit,
           table2, delta3, delta4, delta5, delta6, W, b):
    tcat = jnp.concatenate([table2, delta3, delta4, delta5, delta6], axis=0)
    wb = jnp.concatenate([W.reshape(_D), b, jnp.zeros((15,), jnp.float32)])
    mesh = plsc.VectorSubcoreMesh(core_axis_name="c", subcore_axis_name="s")
    scratch = [pltpu.VMEM((_RPW,), jnp.int32)] * 5 + [
        pltpu.VMEM((_NCH * _CAT,), jnp.int32),
        pltpu.VMEM((_CAT, _D), jnp.float32),
        pltpu.VMEM((_CAT, _D), jnp.float32),
        pltpu.VMEM((_RPW,), jnp.float32),
        pltpu.VMEM((_D + 16,), jnp.float32),
        pltpu.SemaphoreType.DMA,
        pltpu.SemaphoreType.DMA,
    ]
    call = pl.kernel(
        _body,
        out_type=jax.ShapeDtypeStruct((_B,), jnp.float32),
        mesh=mesh,
        scratch_types=scratch,
        compiler_params=pltpu.CompilerParams(needs_layout_passes=False),
    )
    out = call(naics_2_digit, naics_3_digit, naics_4_digit, naics_5_digit,
               naics_6_digit, tcat, wb)
    return out.reshape(_B, 1)


# speedup vs baseline: 1.2496x; 1.2496x over previous
import jax
import jax.numpy as jnp
from jax import lax
from jax.experimental import pallas as pl
from jax.experimental.pallas import tpu as pltpu
from jax.experimental.pallas import tpu_sc as plsc

_B = 16384
_D = 128
_K = _D // 16
_NC = 2
_NS = 16
_NW = _NC * _NS
_RPW = _B // _NW
_C = 64
_NCH = _RPW // _C
_CAT = 5 * _C
_OFFS = (0, 25, 125, 525, 1225)


def _rsqrt_nr(x):
    xi = lax.bitcast_convert_type(x, jnp.int32)
    yi = jnp.int32(0x5F3759DF) - lax.shift_right_logical(xi, 1)
    y = lax.bitcast_convert_type(yi, jnp.float32)
    hx = x * jnp.float32(0.5)
    for _ in range(2):
        y = y * (jnp.float32(1.5) - hx * y * y)
    return y


def _splat(s):
    return lax.broadcast_in_dim(s, (16,), ())


def _body(i2, i3, i4, i5, i6, tcat, wb,
          out_hbm,
          ix0, ix1, ix2, ix3, ix4, ixc,
          tsh, gc0, gc1, out_v, wb_v, semA, semB):
    sid = lax.axis_index("s")
    wid = sid * _NC + lax.axis_index("c")
    base = wid * _RPW

    @pl.when(sid == 0)
    def _():
        pltpu.sync_copy(tcat, tsh)

    idx_refs = (ix0, ix1, ix2, ix3, ix4)
    for idx_hbm, idx_v in zip((i2, i3, i4, i5, i6), idx_refs):
        pltpu.sync_copy(idx_hbm.at[pl.ds(base, _RPW)], idx_v)
    pltpu.sync_copy(wb, wb_v)

    for l in range(5):
        off = jnp.full((16,), _OFFS[l], jnp.int32)
        src = idx_refs[l]

        def mk_cat(i, off=off, src=src):
            ch = i // 4
            q = i % 4
            dst = pl.multiple_of(ch * _CAT + l * _C + q * 16, 16)
            sp = pl.multiple_of(i * 16, 16)
            ixc[pl.ds(dst, 16)] = src[pl.ds(sp, 16)] + off

        plsc.parallel_loop(0, _RPW // 16)(mk_cat)

    gbufs = (gc0, gc1)
    sems = (semA, semB)

    w_regs = [wb_v[pl.ds(k * 16, 16)] for k in range(_K)]
    b_splat = _splat(wb_v[pl.ds(_D, 16)][0])
    lane0 = lax.broadcasted_iota(jnp.int32, (16,), 0) == 0

    def issue(ch):
        p = ch % 2
        return pltpu.async_copy(
            tsh.at[ixc.at[pl.ds(ch * _CAT, _CAT)]], gbufs[p], sems[p])

    plsc.subcore_barrier()
    pending = issue(0)
    for ch in range(_NCH):
        pending.wait()
        if ch + 1 < _NCH:
            pending = issue(ch + 1)
        g = gbufs[ch % 2]
        cb = ch * _C

        def row_body(r, _, g=g, cb=cb):
            u = [g[r, pl.ds(k * 16, 16)] for k in range(_K)]
            acc = u[0] * u[0]
            for k in range(1, _K):
                acc = acc + u[k] * u[k]
            y = _rsqrt_nr(_splat(jnp.sum(acc)))
            for l in range(1, 5):
                u = [y * u[k] + g[l * _C + r, pl.ds(k * 16, 16)]
                     for k in range(_K)]
                acc = u[0] * u[0]
                for k in range(1, _K):
                    acc = acc + u[k] * u[k]
                y = _rsqrt_nr(_splat(jnp.sum(acc)))
            dotv = u[0] * w_regs[0]
            for k in range(1, _K):
                dotv = dotv + u[k] * w_regs[k]
            row_out = y * _splat(jnp.sum(dotv)) + b_splat
            plsc.store_scatter(out_v, [jnp.full((16,), cb + r, jnp.int32)],
                               row_out, mask=lane0)
            return 0

        plsc.parallel_loop(0, _C, carry=jnp.int32(0))(row_body)

    pltpu.sync_copy(out_v, out_hbm.at[pl.ds(base, _RPW)])


def kernel(naics_2_digit, naics_3_digit, naics_4_digit, naics_5_digit, naics_6_digit,
           table2, delta3, delta4, delta5, delta6, W, b):
    tcat = jnp.concatenate([table2, delta3, delta4, delta5, delta6], axis=0)
    wb = jnp.concatenate([W.reshape(_D), b, jnp.zeros((15,), jnp.float32)])
    mesh = plsc.VectorSubcoreMesh(core_axis_name="c", subcore_axis_name="s")
    scratch = [pltpu.VMEM((_RPW,), jnp.int32)] * 5 + [
        pltpu.VMEM((_NCH * _CAT,), jnp.int32),
        pltpu.VMEM_SHARED((2282, _D), jnp.float32),
        pltpu.VMEM((_CAT, _D), jnp.float32),
        pltpu.VMEM((_CAT, _D), jnp.float32),
        pltpu.VMEM((_RPW,), jnp.float32),
        pltpu.VMEM((_D + 16,), jnp.float32),
        pltpu.SemaphoreType.DMA,
        pltpu.SemaphoreType.DMA,
    ]
    call = pl.kernel(
        _body,
        out_type=jax.ShapeDtypeStruct((_B,), jnp.float32),
        mesh=mesh,
        scratch_types=scratch,
        compiler_params=pltpu.CompilerParams(needs_layout_passes=False),
    )
    out = call(naics_2_digit, naics_3_digit, naics_4_digit, naics_5_digit,
               naics_6_digit, tcat, wb)
    return out.reshape(_B, 1)
